# baseline (speedup 1.0000x reference)
import functools

import jax
import jax.numpy as jnp
from jax import lax
from jax.experimental import pallas as pl
from jax.experimental.pallas import tpu as pltpu
from jax.experimental.pallas import tpu_sc as plsc

NUM_ROWS = 16384
SEQ = 50
DIM = 32
B = NUM_ROWS * SEQ

_info = plsc.get_sparse_core_info()
NC = _info.num_cores
NS = _info.num_subcores
NW = NC * NS
BPW = B // NW
CHUNK = 512
NCHUNK = BPW // CHUNK
BUFS = 3
NSTEPS = NCHUNK + 3
NGROUPS = (NSTEPS + BUFS - 1) // BUFS
LANES = 16

_mesh = plsc.VectorSubcoreMesh(core_axis_name="c", subcore_axis_name="s")


@functools.partial(
    pl.kernel,
    mesh=_mesh,
    out_type=jax.ShapeDtypeStruct((SEQ, DIM, NUM_ROWS), jnp.float32),
    scratch_types=[
        pltpu.VMEM((BPW,), jnp.int32),
        pltpu.VMEM((BUFS, CHUNK, DIM), jnp.float32),
        pltpu.VMEM((BUFS, DIM, CHUNK), jnp.float32),
        pltpu.SemaphoreType.DMA,
        pltpu.SemaphoreType.DMA,
        pltpu.SemaphoreType.DMA,
        pltpu.SemaphoreType.DMA,
        pltpu.SemaphoreType.DMA,
        pltpu.SemaphoreType.DMA,
    ],
    compiler_params=pltpu.CompilerParams(use_tc_tiling_on_sc=False,
                                         needs_layout_passes=False),
)
def _embed_sc(idx_hbm, tbl_hbm, out_hbm, idx_v, rows_v, trans_v,
              g0, g1, g2, w0, w1, w2):
    wid = lax.axis_index("s") * NC + lax.axis_index("c")
    base = wid * BPW
    gsem = (g0, g1, g2)
    wsem = (w0, w1, w2)

    pltpu.sync_copy(idx_hbm.at[pl.ds(base, BPW)], idx_v)

    iota = lax.iota(jnp.int32, LANES)
    cols = [jnp.full((LANES,), d, jnp.int32) for d in range(DIM)]

    def chunk_pos(k):
        f0 = base + k * CHUNK
        return lax.div(f0, NUM_ROWS), lax.rem(f0, NUM_ROWS)

    def gather(k, b):
        return pltpu.make_async_copy(
            tbl_hbm.at[idx_v.at[pl.ds(k * CHUNK, CHUNK)]],
            rows_v.at[b], gsem[b])

    def write(k, b):
        s, b0 = chunk_pos(k)
        return pltpu.make_async_copy(
            trans_v.at[b], out_hbm.at[s, :, pl.ds(b0, CHUNK)], wsem[b])

    def transpose(b):
        src = rows_v.at[b]
        dst = trans_v.at[b]

        def body(jg, _):
            rows = jg * LANES + iota
            for d in range(DIM):
                v = plsc.load_gather(src, [rows, cols[d]])
                dst[d, pl.ds(jg * LANES, LANES)] = v
            return _

        lax.fori_loop(0, CHUNK // LANES, body, 0)

    def step(k, j):
        jp = (j + 1) % BUFS

        @pl.when(jnp.logical_and(k >= BUFS, k <= NCHUNK - 1 + BUFS))
        def _():
            write(k - BUFS, j).wait()

        @pl.when(k <= NCHUNK - 1)
        def _():
            gather(k, j).start()

        @pl.when(jnp.logical_and(k >= 2, k <= NCHUNK + 1))
        def _():
            gather(k - 2, jp).wait()
            transpose(jp)
            write(k - 2, jp).start()

    def group(g, carry):
        for j in range(BUFS):
            step(g * BUFS + j, j)
        return carry

    lax.fori_loop(0, NGROUPS, group, 0)


def kernel(token_ids, weight):
    flat = token_ids.T.reshape(B)
    out = _embed_sc(flat, weight)
    return jnp.transpose(out, (2, 0, 1))


# speedup vs baseline: 1.3276x; 1.3276x over previous
import functools

import jax
import jax.numpy as jnp
from jax import lax
from jax.experimental import pallas as pl
from jax.experimental.pallas import tpu as pltpu
from jax.experimental.pallas import tpu_sc as plsc

NUM_ROWS = 16384
SEQ = 50
DIM = 32
B = NUM_ROWS * SEQ

_info = plsc.get_sparse_core_info()
NC = _info.num_cores
NS = _info.num_subcores
NW = NC * NS
BPW = B // NW
CHUNK = 1024
NCHUNK = BPW // CHUNK
BUFS = 3
LEAD = 2

_mesh = plsc.VectorSubcoreMesh(core_axis_name="c", subcore_axis_name="s")


@functools.partial(
    pl.kernel,
    mesh=_mesh,
    out_type=jax.ShapeDtypeStruct((SEQ, NUM_ROWS, DIM), jnp.float32),
    scratch_types=[
        pltpu.VMEM((BPW,), jnp.int32),
        pltpu.VMEM((BUFS, CHUNK, DIM), jnp.float32),
        pltpu.SemaphoreType.DMA,
        pltpu.SemaphoreType.DMA,
        pltpu.SemaphoreType.DMA,
        pltpu.SemaphoreType.DMA,
        pltpu.SemaphoreType.DMA,
        pltpu.SemaphoreType.DMA,
    ],
    compiler_params=pltpu.CompilerParams(use_tc_tiling_on_sc=False),
)
def _embed_sc(idx_hbm, tbl_hbm, out_hbm, idx_v, rows_v,
              g0, g1, g2, w0, w1, w2):
    wid = lax.axis_index("s") * NC + lax.axis_index("c")
    base = wid * BPW
    gsem = (g0, g1, g2)
    wsem = (w0, w1, w2)

    pltpu.sync_copy(idx_hbm.at[pl.ds(base, BPW)], idx_v)

    def gather(k):
        b = k % BUFS
        return pltpu.make_async_copy(
            tbl_hbm.at[idx_v.at[pl.ds(k * CHUNK, CHUNK)]],
            rows_v.at[b], gsem[b])

    def write(k):
        b = k % BUFS
        f0 = base + k * CHUNK
        s = lax.div(f0, NUM_ROWS)
        b0 = lax.rem(f0, NUM_ROWS)
        return pltpu.make_async_copy(
            rows_v.at[b], out_hbm.at[s, pl.ds(b0, CHUNK)], wsem[b])

    for k in range(LEAD):
        gather(k).start()
    for k in range(NCHUNK):
        nxt = k + LEAD
        if nxt < NCHUNK:
            if nxt >= BUFS:
                write(nxt - BUFS).wait()
            gather(nxt).start()
        gather(k).wait()
        write(k).start()
    for k in range(NCHUNK - BUFS, NCHUNK):
        write(k).wait()


def kernel(token_ids, weight):
    flat = token_ids.T.reshape(B)
    out = _embed_sc(flat, weight)
    return jnp.transpose(out, (1, 0, 2))


# speedup vs baseline: 1.3287x; 1.0009x over previous
import functools

import jax
import jax.numpy as jnp
from jax import lax
from jax.experimental import pallas as pl
from jax.experimental.pallas import tpu as pltpu
from jax.experimental.pallas import tpu_sc as plsc

NUM_ROWS = 16384
SEQ = 50
DIM = 32
B = NUM_ROWS * SEQ

_info = plsc.get_sparse_core_info()
NC = _info.num_cores
NS = _info.num_subcores
NW = NC * NS
BPW = B // NW
CHUNK = 1024
NCHUNK = BPW // CHUNK
BUFS = 3
LEAD = 2

_mesh = plsc.VectorSubcoreMesh(core_axis_name="c", subcore_axis_name="s")

_DT_CC = 2048
_DT_ITEMS = 7 * (NUM_ROWS // _DT_CC)


@functools.partial(
    pl.kernel,
    mesh=_mesh,
    out_type=jax.ShapeDtypeStruct((B,), jnp.int32),
    scratch_types=[
        pltpu.VMEM((8, _DT_CC), jnp.int32),
    ],
    compiler_params=pltpu.CompilerParams(use_tc_tiling_on_sc=True),
)
def _detile_sc(tok_hbm, out_hbm, buf_v):
    wid = lax.axis_index("s") * NC + lax.axis_index("c")

    def do_item(it):
        tr = it // (NUM_ROWS // _DT_CC)
        cc = it % (NUM_ROWS // _DT_CC)
        c0 = cc * _DT_CC
        pltpu.sync_copy(tok_hbm.at[pl.ds(tr * 8, 8), pl.ds(c0, _DT_CC)],
                        buf_v)
        for r in range(8):
            s = tr * 8 + r

            @pl.when(s < SEQ)
            def _():
                pltpu.sync_copy(
                    buf_v.at[r],
                    out_hbm.at[pl.ds(s * NUM_ROWS + c0, _DT_CC)])

    for i in range(2):
        it = wid + i * NW

        @pl.when(it < _DT_ITEMS)
        def _():
            do_item(it)


@functools.partial(
    pl.kernel,
    mesh=_mesh,
    out_type=jax.ShapeDtypeStruct((SEQ, NUM_ROWS, DIM), jnp.float32),
    scratch_types=[
        pltpu.VMEM((BPW,), jnp.int32),
        pltpu.VMEM((BUFS, CHUNK, DIM), jnp.float32),
        pltpu.SemaphoreType.DMA,
        pltpu.SemaphoreType.DMA,
        pltpu.SemaphoreType.DMA,
        pltpu.SemaphoreType.DMA,
        pltpu.SemaphoreType.DMA,
        pltpu.SemaphoreType.DMA,
    ],
    compiler_params=pltpu.CompilerParams(use_tc_tiling_on_sc=False),
)
def _embed_sc(idx_hbm, tbl_hbm, out_hbm, idx_v, rows_v,
              g0, g1, g2, w0, w1, w2):
    wid = lax.axis_index("s") * NC + lax.axis_index("c")
    base = wid * BPW
    gsem = (g0, g1, g2)
    wsem = (w0, w1, w2)

    pltpu.sync_copy(idx_hbm.at[pl.ds(base, BPW)], idx_v)

    def gather(k):
        b = k % BUFS
        return pltpu.make_async_copy(
            tbl_hbm.at[idx_v.at[pl.ds(k * CHUNK, CHUNK)]],
            rows_v.at[b], gsem[b])

    def write(k):
        b = k % BUFS
        f0 = base + k * CHUNK
        s = lax.div(f0, NUM_ROWS)
        b0 = lax.rem(f0, NUM_ROWS)
        return pltpu.make_async_copy(
            rows_v.at[b], out_hbm.at[s, pl.ds(b0, CHUNK)], wsem[b])

    for k in range(LEAD):
        gather(k).start()
    for k in range(NCHUNK):
        nxt = k + LEAD
        if nxt < NCHUNK:
            if nxt >= BUFS:
                write(nxt - BUFS).wait()
            gather(nxt).start()
        gather(k).wait()
        write(k).start()
    for k in range(NCHUNK - BUFS, NCHUNK):
        write(k).wait()


def kernel(token_ids, weight):
    flat = _detile_sc(token_ids.T)
    out = _embed_sc(flat, weight)
    return jnp.transpose(out, (1, 0, 2))
